# Initial kernel scaffold; baseline (speedup 1.0000x reference)
import jax
import jax.numpy as jnp
from jax.experimental import pallas as pl


def kernel(entity_pairs, train_edges, labels, entity2edges, edge2entities, edge2relation, relation_features_base, agg0_W, agg0_b, agg1_W, agg1_b, lin_out_W, lin_out_b, mlp_W1, mlp_b1, mlp_W2, mlp_b2, mlp_W3, mlp_b3, mlp_W4, mlp_b4, t_rand, eps):
    raise NotImplementedError("write your pallas kernel here")


# speedup vs baseline: 10.8594x; 10.8594x over previous
import functools
import jax
import jax.numpy as jnp
from jax import lax
from jax.experimental import pallas as pl
from jax.experimental.pallas import tpu as pltpu
from jax.experimental.pallas import tpu_sc as plsc

N_REL = 32
N_ENT = 100000
N_EDGE = 1600000
S = 16
H = 128
B = 1024
W_CFM = 128
SIGMA = 0.1

NITEMS = 2 * B
NW = 32
IPW = NITEMS // NW
HB = 48


def _iota16():
    return lax.broadcasted_iota(jnp.int32, (16,), 0)


def _sc_kernel_body(ents_hbm, te_hbm, e2e_hbm, ent0_hbm, ent1_hbm, rel_hbm,
                    cnt_hbm, m1_hbm,
                    ents_v, te_v, e1, e1flat, ents2a, ents2b, r1,
                    e2, tflat, r2t, cntbuf, m1buf, sem):
    wid = lax.axis_index("s") * 2 + lax.axis_index("c")
    ibase = wid * IPW

    pltpu.sync_copy(ents_hbm.at[pl.ds(ibase, IPW)], ents_v)
    pltpu.sync_copy(te_hbm.at[pl.ds(ibase, IPW)], te_v)

    pltpu.async_copy(e2e_hbm.at[ents_v], e1, sem).wait()

    for j in range(IPW):
        e1flat[pl.ds(j * 16, 16)] = e1[j, :]

    for c in range(IPW * 16 // 128):
        idx = e1flat.at[pl.ds(c * 128, 128)]
        pltpu.async_copy(ent0_hbm.at[idx], ents2a.at[pl.ds(c * 128, 128)], sem).wait()
        pltpu.async_copy(ent1_hbm.at[idx], ents2b.at[pl.ds(c * 128, 128)], sem).wait()
        pltpu.async_copy(rel_hbm.at[idx], r1.at[pl.ds(c * 128, 128)], sem).wait()

    def zero_body(z, _):
        cntbuf[pl.ds(z * 16, 16)] = jnp.zeros((16,), jnp.float32)
        return _
    lax.fori_loop(0, IPW * S * HB // 16, zero_body, 0)

    iota = _iota16()

    def item_body(i, _):
        pltpu.async_copy(e2e_hbm.at[ents2a.at[pl.ds(i * 16, 16)]],
                         e2.at[pl.ds(0, 16)], sem).wait()
        pltpu.async_copy(e2e_hbm.at[ents2b.at[pl.ds(i * 16, 16)]],
                         e2.at[pl.ds(16, 16)], sem).wait()

        for k in range(32):
            e_bit = 0 if k < 16 else 1
            s_id = k % 16
            vec = e2[k, :]
            tgt = iota * 16 + (e_bit * 256 + s_id)
            plsc.store_scatter(tflat, [tgt], vec)

        for c in range(4):
            pltpu.async_copy(rel_hbm.at[tflat.at[pl.ds(c * 128, 128)]],
                             r2t.at[pl.ds(c * 128, 16 * 8)], sem).wait()

        te_b = plsc.load_gather(te_v, [iota * 0 + i])

        base = i * (S * HB)
        for k in range(32):
            edges_vec = tflat[pl.ds(k * 16, 16)]
            rvec = r2t[pl.ds(k * 16, 16)]
            m = jnp.where(edges_vec != te_b, 1.0, 0.0).astype(jnp.float32)
            plsc.addupdate_scatter(cntbuf, [base + iota * HB + rvec], m)

        rvec1 = r1[pl.ds(i * 16, 16)]
        plsc.addupdate_scatter(cntbuf, [base + iota * HB + rvec1],
                               jnp.full((16,), 32.0, jnp.float32))

        e1vec = e1flat[pl.ds(i * 16, 16)]
        m1buf[pl.ds(i * 16, 16)] = jnp.where(e1vec != te_b, 1.0, 0.0).astype(jnp.float32)
        return _

    lax.fori_loop(0, IPW, item_body, 0)

    pltpu.sync_copy(cntbuf, cnt_hbm.at[pl.ds(wid * (IPW * S * HB), IPW * S * HB)])
    pltpu.sync_copy(m1buf, m1_hbm.at[pl.ds(wid * (IPW * 16), IPW * 16)])


def _make_sc_call():
    mesh = plsc.VectorSubcoreMesh(core_axis_name="c", subcore_axis_name="s")
    return pl.kernel(
        _sc_kernel_body,
        mesh=mesh,
        compiler_params=pltpu.CompilerParams(needs_layout_passes=False,
                                             use_tc_tiling_on_sc=False),
        out_type=[
            jax.ShapeDtypeStruct((NITEMS * S * HB,), jnp.float32),
            jax.ShapeDtypeStruct((NITEMS * 16,), jnp.float32),
        ],
        scratch_types=[
            pltpu.VMEM((IPW,), jnp.int32),
            pltpu.VMEM((IPW,), jnp.int32),
            pltpu.VMEM((IPW, 16), jnp.int32),
            pltpu.VMEM((IPW * 16,), jnp.int32),
            pltpu.VMEM((IPW * 16,), jnp.int32),
            pltpu.VMEM((IPW * 16,), jnp.int32),
            pltpu.VMEM((IPW * 16,), jnp.int32),
            pltpu.VMEM((32, 16), jnp.int32),
            pltpu.VMEM((512,), jnp.int32),
            pltpu.VMEM((512,), jnp.int32),
            pltpu.VMEM((IPW * S * HB,), jnp.float32),
            pltpu.VMEM((IPW * 16,), jnp.float32),
            pltpu.SemaphoreType.DMA,
        ],
    )


def _agg_tc_kernel(cnt_ref, m1_ref, rf_ref, w0_ref, b0_ref, w1_ref, b1_ref, out_ref):
    m = jnp.dot(rf_ref[...], w0_ref[...], preferred_element_type=jnp.float32) * (1.0 / 32.0)
    m48 = jnp.concatenate([m, jnp.zeros((HB - N_REL, H), jnp.float32)], axis=0)
    cnt = cnt_ref[...]
    new1 = jnp.dot(cnt, m48, preferred_element_type=jnp.float32) + b0_ref[...]
    new1 = jnp.maximum(new1, 0.0)
    blk = new1.shape[0] // S
    new1 = new1.reshape(blk, S, H) * m1_ref[...][:, :, None]
    pooled = jnp.sum(new1, axis=1) * (1.0 / S)
    out_ref[...] = jnp.dot(pooled, w1_ref[...], preferred_element_type=jnp.float32) + b1_ref[...]


def _selu(x):
    alpha = 1.6732632423543772848170429916717
    scale = 1.0507009873554804934193349852946
    return scale * jnp.where(x > 0, x, alpha * (jnp.exp(x) - 1.0))


def _head_tc_kernel(agg_ref, lw_ref, lb_ref, w1_ref, b1_ref, w2_ref, b2_ref,
                    w3_ref, b3_ref, w4_ref, b4_ref, t_ref, eps_ref,
                    scores_ref, loss_ref):
    agg = agg_ref[...]
    x0 = agg[:B, :]
    x1 = agg[B:, :]
    output = jnp.dot(jnp.concatenate([x0, x1], axis=1), lw_ref[...],
                     preferred_element_type=jnp.float32) + lb_ref[...]
    t = t_ref[...][:, None]
    mu_t = t * x1 + (1.0 - t) * x0
    xt = mu_t + SIGMA * eps_ref[...]
    ut = x1 - x0
    hcat = jnp.concatenate([xt, t], axis=1)
    v = _selu(jnp.dot(hcat, w1_ref[...], preferred_element_type=jnp.float32) + b1_ref[...])
    v = _selu(jnp.dot(v, w2_ref[...], preferred_element_type=jnp.float32) + b2_ref[...])
    v = _selu(jnp.dot(v, w3_ref[...], preferred_element_type=jnp.float32) + b3_ref[...])
    vt_pred = jnp.dot(v, w4_ref[...], preferred_element_type=jnp.float32) + b4_ref[...]
    loss_ref[0, 0] = jnp.mean((vt_pred - ut) ** 2)
    scores_ref[...] = output * vt_pred


@jax.jit
def kernel(entity_pairs, train_edges, labels, entity2edges, edge2entities,
           edge2relation, relation_features_base, agg0_W, agg0_b, agg1_W,
           agg1_b, lin_out_W, lin_out_b, mlp_W1, mlp_b1, mlp_W2, mlp_b2,
           mlp_W3, mlp_b3, mlp_W4, mlp_b4, t_rand, eps):
    del labels

    ents = jnp.concatenate([entity_pairs[:, 0], entity_pairs[:, 1]])
    te2 = jnp.concatenate([train_edges, train_edges])
    ent0 = edge2entities[:, 0]
    ent1 = edge2entities[:, 1]

    cnt_flat, m1_flat = _make_sc_call()(ents, te2, entity2edges, ent0, ent1,
                                        edge2relation)

    cnt = cnt_flat.reshape(NITEMS * S, HB)
    m1 = m1_flat.reshape(NITEMS, 16)

    blk = 256
    agg = pl.pallas_call(
        _agg_tc_kernel,
        grid=(NITEMS // blk,),
        in_specs=[
            pl.BlockSpec((blk * S, HB), lambda i: (i, 0)),
            pl.BlockSpec((blk, 16), lambda i: (i, 0)),
            pl.BlockSpec((N_REL, H), lambda i: (0, 0)),
            pl.BlockSpec((H, H), lambda i: (0, 0)),
            pl.BlockSpec((H,), lambda i: (0,)),
            pl.BlockSpec((H, N_REL), lambda i: (0, 0)),
            pl.BlockSpec((N_REL,), lambda i: (0,)),
        ],
        out_specs=pl.BlockSpec((blk, N_REL), lambda i: (i, 0)),
        out_shape=jax.ShapeDtypeStruct((NITEMS, N_REL), jnp.float32),
    )(cnt, m1, relation_features_base, agg0_W, agg0_b, agg1_W, agg1_b)

    scores, loss = pl.pallas_call(
        _head_tc_kernel,
        out_shape=(
            jax.ShapeDtypeStruct((B, N_REL), jnp.float32),
            jax.ShapeDtypeStruct((1, 1), jnp.float32),
        ),
        out_specs=(
            pl.BlockSpec(memory_space=pltpu.VMEM),
            pl.BlockSpec(memory_space=pltpu.SMEM),
        ),
    )(agg, lin_out_W, lin_out_b, mlp_W1, mlp_b1, mlp_W2, mlp_b2,
      mlp_W3, mlp_b3, mlp_W4, mlp_b4, t_rand, eps)

    return scores, loss[0, 0]


# speedup vs baseline: 17.4621x; 1.6080x over previous
import functools
import jax
import jax.numpy as jnp
from jax import lax
from jax.experimental import pallas as pl
from jax.experimental.pallas import tpu as pltpu
from jax.experimental.pallas import tpu_sc as plsc

N_REL = 32
N_ENT = 100000
N_EDGE = 1600000
S = 16
H = 128
B = 1024
W_CFM = 128
SIGMA = 0.1

NITEMS = 2 * B
NW = 32
IPW = NITEMS // NW
HB = 48


def _iota16():
    return lax.broadcasted_iota(jnp.int32, (16,), 0)


def _sc_kernel_body(ents_hbm, te_hbm, e2e_hbm, ent0_hbm, ent1_hbm, rel_hbm,
                    cnt_hbm, m1_hbm,
                    ents_v, te_v, e1, e1flat, ents2a, ents2b, r1,
                    e2a, e2b, tflat, r2t, cntbuf, m1buf,
                    sem, sema, semb, semr):
    wid = lax.axis_index("s") * 2 + lax.axis_index("c")
    ibase = wid * IPW

    pltpu.sync_copy(ents_hbm.at[pl.ds(ibase, IPW)], ents_v)
    pltpu.sync_copy(te_hbm.at[pl.ds(ibase, IPW)], te_v)

    pltpu.async_copy(e2e_hbm.at[ents_v], e1, sem).wait()

    for j in range(IPW):
        e1flat[pl.ds(j * 16, 16)] = e1[j, :]

    handles = []
    for c in range(IPW * 16 // 128):
        idx = e1flat.at[pl.ds(c * 128, 128)]
        handles.append(pltpu.async_copy(ent0_hbm.at[idx],
                                        ents2a.at[pl.ds(c * 128, 128)], sem))
        handles.append(pltpu.async_copy(ent1_hbm.at[idx],
                                        ents2b.at[pl.ds(c * 128, 128)], sem))
        handles.append(pltpu.async_copy(rel_hbm.at[idx],
                                        r1.at[pl.ds(c * 128, 128)], sem))
    for h in handles:
        h.wait()

    def zero_body(z, _):
        cntbuf[pl.ds(z * 16, 16)] = jnp.zeros((16,), jnp.float32)
        return _
    lax.fori_loop(0, IPW * S * HB // 16, zero_body, 0)

    iota = _iota16()

    def fetch_e2(i, buf, bsem):
        h0 = pltpu.async_copy(e2e_hbm.at[ents2a.at[pl.ds(i * 16, 16)]],
                              buf.at[pl.ds(0, 16)], bsem)
        h1 = pltpu.async_copy(e2e_hbm.at[ents2b.at[pl.ds(i * 16, 16)]],
                              buf.at[pl.ds(16, 16)], bsem)
        return h0, h1

    def process(i, buf, h0, h1):
        h0.wait()
        h1.wait()
        for k in range(32):
            e_bit = 0 if k < 16 else 1
            s_id = k % 16
            vec = buf[k, :]
            tgt = iota * 16 + (e_bit * 256 + s_id)
            plsc.store_scatter(tflat, [tgt], vec)

        rh = [pltpu.async_copy(rel_hbm.at[tflat.at[pl.ds(c * 128, 128)]],
                               r2t.at[pl.ds(c * 128, 128)], semr)
              for c in range(4)]

        te_b = plsc.load_gather(te_v, [iota * 0 + i])
        base = i * (S * HB)

        rvec1 = r1[pl.ds(i * 16, 16)]
        plsc.addupdate_scatter(cntbuf, [base + iota * HB + rvec1],
                               jnp.full((16,), 32.0, jnp.float32))

        e1vec = e1flat[pl.ds(i * 16, 16)]
        m1buf[pl.ds(i * 16, 16)] = jnp.where(e1vec != te_b, 1.0, 0.0).astype(jnp.float32)

        for h in rh:
            h.wait()
        for k in range(32):
            edges_vec = tflat[pl.ds(k * 16, 16)]
            rvec = r2t[pl.ds(k * 16, 16)]
            m = jnp.where(edges_vec != te_b, 1.0, 0.0).astype(jnp.float32)
            plsc.addupdate_scatter(cntbuf, [base + iota * HB + rvec], m)

    def loop_body(j, _):
        i = j * 2
        ha0, ha1 = fetch_e2(i, e2a, sema)
        hb0, hb1 = fetch_e2(i + 1, e2b, semb)
        process(i, e2a, ha0, ha1)
        process(i + 1, e2b, hb0, hb1)
        return _

    lax.fori_loop(0, IPW // 2, loop_body, 0)

    pltpu.sync_copy(cntbuf, cnt_hbm.at[pl.ds(wid * (IPW * S * HB), IPW * S * HB)])
    pltpu.sync_copy(m1buf, m1_hbm.at[pl.ds(wid * (IPW * 16), IPW * 16)])


def _make_sc_call():
    mesh = plsc.VectorSubcoreMesh(core_axis_name="c", subcore_axis_name="s")
    return pl.kernel(
        _sc_kernel_body,
        mesh=mesh,
        compiler_params=pltpu.CompilerParams(needs_layout_passes=False,
                                             use_tc_tiling_on_sc=False),
        out_type=[
            jax.ShapeDtypeStruct((NITEMS * S * HB,), jnp.float32),
            jax.ShapeDtypeStruct((NITEMS * 16,), jnp.float32),
        ],
        scratch_types=[
            pltpu.VMEM((IPW,), jnp.int32),
            pltpu.VMEM((IPW,), jnp.int32),
            pltpu.VMEM((IPW, 16), jnp.int32),
            pltpu.VMEM((IPW * 16,), jnp.int32),
            pltpu.VMEM((IPW * 16,), jnp.int32),
            pltpu.VMEM((IPW * 16,), jnp.int32),
            pltpu.VMEM((IPW * 16,), jnp.int32),
            pltpu.VMEM((32, 16), jnp.int32),
            pltpu.VMEM((32, 16), jnp.int32),
            pltpu.VMEM((512,), jnp.int32),
            pltpu.VMEM((512,), jnp.int32),
            pltpu.VMEM((IPW * S * HB,), jnp.float32),
            pltpu.VMEM((IPW * 16,), jnp.float32),
            pltpu.SemaphoreType.DMA,
            pltpu.SemaphoreType.DMA,
            pltpu.SemaphoreType.DMA,
            pltpu.SemaphoreType.DMA,
        ],
    )


def _agg_tc_kernel(cnt_ref, m1_ref, rf_ref, w0_ref, b0_ref, w1_ref, b1_ref, out_ref):
    m = jnp.dot(rf_ref[...], w0_ref[...], preferred_element_type=jnp.float32) * (1.0 / 32.0)
    m48 = jnp.concatenate([m, jnp.zeros((HB - N_REL, H), jnp.float32)], axis=0)
    cnt = cnt_ref[...]
    new1 = jnp.dot(cnt, m48, preferred_element_type=jnp.float32) + b0_ref[...]
    new1 = jnp.maximum(new1, 0.0)
    blk = new1.shape[0] // S
    new1 = new1.reshape(blk, S, H) * m1_ref[...][:, :, None]
    pooled = jnp.sum(new1, axis=1) * (1.0 / S)
    out_ref[...] = jnp.dot(pooled, w1_ref[...], preferred_element_type=jnp.float32) + b1_ref[...]


def _selu(x):
    alpha = 1.6732632423543772848170429916717
    scale = 1.0507009873554804934193349852946
    return scale * jnp.where(x > 0, x, alpha * (jnp.exp(x) - 1.0))


def _head_tc_kernel(agg_ref, lw_ref, lb_ref, w1_ref, b1_ref, w2_ref, b2_ref,
                    w3_ref, b3_ref, w4_ref, b4_ref, t_ref, eps_ref,
                    scores_ref, loss_ref):
    agg = agg_ref[...]
    x0 = agg[:B, :]
    x1 = agg[B:, :]
    output = jnp.dot(jnp.concatenate([x0, x1], axis=1), lw_ref[...],
                     preferred_element_type=jnp.float32) + lb_ref[...]
    t = t_ref[...][:, None]
    mu_t = t * x1 + (1.0 - t) * x0
    xt = mu_t + SIGMA * eps_ref[...]
    ut = x1 - x0
    hcat = jnp.concatenate([xt, t], axis=1)
    v = _selu(jnp.dot(hcat, w1_ref[...], preferred_element_type=jnp.float32) + b1_ref[...])
    v = _selu(jnp.dot(v, w2_ref[...], preferred_element_type=jnp.float32) + b2_ref[...])
    v = _selu(jnp.dot(v, w3_ref[...], preferred_element_type=jnp.float32) + b3_ref[...])
    vt_pred = jnp.dot(v, w4_ref[...], preferred_element_type=jnp.float32) + b4_ref[...]
    loss_ref[0, 0] = jnp.mean((vt_pred - ut) ** 2)
    scores_ref[...] = output * vt_pred


@jax.jit
def kernel(entity_pairs, train_edges, labels, entity2edges, edge2entities,
           edge2relation, relation_features_base, agg0_W, agg0_b, agg1_W,
           agg1_b, lin_out_W, lin_out_b, mlp_W1, mlp_b1, mlp_W2, mlp_b2,
           mlp_W3, mlp_b3, mlp_W4, mlp_b4, t_rand, eps):
    del labels

    ents = jnp.concatenate([entity_pairs[:, 0], entity_pairs[:, 1]])
    te2 = jnp.concatenate([train_edges, train_edges])
    ent0 = edge2entities[:, 0]
    ent1 = edge2entities[:, 1]

    cnt_flat, m1_flat = _make_sc_call()(ents, te2, entity2edges, ent0, ent1,
                                        edge2relation)

    cnt = cnt_flat.reshape(NITEMS * S, HB)
    m1 = m1_flat.reshape(NITEMS, 16)

    blk = 256
    agg = pl.pallas_call(
        _agg_tc_kernel,
        grid=(NITEMS // blk,),
        in_specs=[
            pl.BlockSpec((blk * S, HB), lambda i: (i, 0)),
            pl.BlockSpec((blk, 16), lambda i: (i, 0)),
            pl.BlockSpec((N_REL, H), lambda i: (0, 0)),
            pl.BlockSpec((H, H), lambda i: (0, 0)),
            pl.BlockSpec((H,), lambda i: (0,)),
            pl.BlockSpec((H, N_REL), lambda i: (0, 0)),
            pl.BlockSpec((N_REL,), lambda i: (0,)),
        ],
        out_specs=pl.BlockSpec((blk, N_REL), lambda i: (i, 0)),
        out_shape=jax.ShapeDtypeStruct((NITEMS, N_REL), jnp.float32),
    )(cnt, m1, relation_features_base, agg0_W, agg0_b, agg1_W, agg1_b)

    scores, loss = pl.pallas_call(
        _head_tc_kernel,
        out_shape=(
            jax.ShapeDtypeStruct((B, N_REL), jnp.float32),
            jax.ShapeDtypeStruct((1, 1), jnp.float32),
        ),
        out_specs=(
            pl.BlockSpec(memory_space=pltpu.VMEM),
            pl.BlockSpec(memory_space=pltpu.SMEM),
        ),
    )(agg, lin_out_W, lin_out_b, mlp_W1, mlp_b1, mlp_W2, mlp_b2,
      mlp_W3, mlp_b3, mlp_W4, mlp_b4, t_rand, eps)

    return scores, loss[0, 0]


# speedup vs baseline: 21.9287x; 1.2558x over previous
import functools
import jax
import jax.numpy as jnp
from jax import lax
from jax.experimental import pallas as pl
from jax.experimental.pallas import tpu as pltpu
from jax.experimental.pallas import tpu_sc as plsc

N_REL = 32
N_ENT = 100000
N_EDGE = 1600000
S = 16
H = 128
B = 1024
W_CFM = 128
SIGMA = 0.1

NITEMS = 2 * B
NW = 32
IPW = NITEMS // NW
HB = 48


def _iota16():
    return lax.broadcasted_iota(jnp.int32, (16,), 0)


def _sc_kernel_body(ents_hbm, te_hbm, e2e_hbm, ent0_hbm, ent1_hbm, rel_hbm,
                    cnt_hbm, m1_hbm,
                    ents_v, te_v, e1, e1flat, ents2a, ents2b, r1,
                    e2a, e2b, e2c, e2d, tfa, tfb, tfc, tfd,
                    r2a, r2b, r2c, r2d, cntbuf, m1buf,
                    sem, sema, semb, semc, semd, semra, semrb, semrc, semrd):
    wid = lax.axis_index("s") * 2 + lax.axis_index("c")
    ibase = wid * IPW

    pltpu.sync_copy(ents_hbm.at[pl.ds(ibase, IPW)], ents_v)
    pltpu.sync_copy(te_hbm.at[pl.ds(ibase, IPW)], te_v)

    pltpu.async_copy(e2e_hbm.at[ents_v], e1, sem).wait()

    for j in range(IPW):
        e1flat[pl.ds(j * 16, 16)] = e1[j, :]

    handles = []
    for c in range(IPW * 16 // 128):
        idx = e1flat.at[pl.ds(c * 128, 128)]
        handles.append(pltpu.async_copy(ent0_hbm.at[idx],
                                        ents2a.at[pl.ds(c * 128, 128)], sem))
        handles.append(pltpu.async_copy(ent1_hbm.at[idx],
                                        ents2b.at[pl.ds(c * 128, 128)], sem))
        handles.append(pltpu.async_copy(rel_hbm.at[idx],
                                        r1.at[pl.ds(c * 128, 128)], sem))
    for h in handles:
        h.wait()

    def zero_body(z, _):
        cntbuf[pl.ds(z * 16, 16)] = jnp.zeros((16,), jnp.float32)
        return _
    lax.fori_loop(0, IPW * S * HB // 16, zero_body, 0)

    iota = _iota16()

    def fetch_e2(i, buf, bsem):
        h0 = pltpu.async_copy(e2e_hbm.at[ents2a.at[pl.ds(i * 16, 16)]],
                              buf.at[pl.ds(0, 16)], bsem)
        h1 = pltpu.async_copy(e2e_hbm.at[ents2b.at[pl.ds(i * 16, 16)]],
                              buf.at[pl.ds(16, 16)], bsem)
        return h0, h1

    def transpose_fire(buf, h0, h1, tf, r2, rsem):
        h0.wait()
        h1.wait()
        for k in range(32):
            e_bit = 0 if k < 16 else 1
            s_id = k % 16
            vec = buf[k, :]
            tgt = iota * 16 + (e_bit * 256 + s_id)
            plsc.store_scatter(tf, [tgt], vec)
        return [pltpu.async_copy(rel_hbm.at[tf.at[pl.ds(c * 128, 128)]],
                                 r2.at[pl.ds(c * 128, 128)], rsem)
                for c in range(4)]

    def hist(i, tf, r2, rh):
        te_b = plsc.load_gather(te_v, [iota * 0 + i])
        base = i * (S * HB)

        rvec1 = r1[pl.ds(i * 16, 16)]
        plsc.addupdate_scatter(cntbuf, [base + iota * HB + rvec1],
                               jnp.full((16,), 32.0, jnp.float32))

        e1vec = e1flat[pl.ds(i * 16, 16)]
        m1buf[pl.ds(i * 16, 16)] = jnp.where(e1vec != te_b, 1.0, 0.0).astype(jnp.float32)

        for h in rh:
            h.wait()
        for k in range(32):
            edges_vec = tf[pl.ds(k * 16, 16)]
            rvec = r2[pl.ds(k * 16, 16)]
            m = jnp.where(edges_vec != te_b, 1.0, 0.0).astype(jnp.float32)
            plsc.addupdate_scatter(cntbuf, [base + iota * HB + rvec], m)

    e2s = (e2a, e2b, e2c, e2d)
    tfs = (tfa, tfb, tfc, tfd)
    r2s = (r2a, r2b, r2c, r2d)
    esems = (sema, semb, semc, semd)
    rsems = (semra, semrb, semrc, semrd)

    def loop_body(j, _):
        i = j * 4
        eh = [fetch_e2(i + b, e2s[b], esems[b]) for b in range(4)]
        rhs = [transpose_fire(e2s[b], *eh[b], tfs[b], r2s[b], rsems[b])
               for b in range(4)]
        for b in range(4):
            hist(i + b, tfs[b], r2s[b], rhs[b])
        return _

    lax.fori_loop(0, IPW // 4, loop_body, 0)

    pltpu.sync_copy(cntbuf, cnt_hbm.at[pl.ds(wid * (IPW * S * HB), IPW * S * HB)])
    pltpu.sync_copy(m1buf, m1_hbm.at[pl.ds(wid * (IPW * 16), IPW * 16)])


def _make_sc_call():
    mesh = plsc.VectorSubcoreMesh(core_axis_name="c", subcore_axis_name="s")
    return pl.kernel(
        _sc_kernel_body,
        mesh=mesh,
        compiler_params=pltpu.CompilerParams(needs_layout_passes=False,
                                             use_tc_tiling_on_sc=False),
        out_type=[
            jax.ShapeDtypeStruct((NITEMS * S * HB,), jnp.float32),
            jax.ShapeDtypeStruct((NITEMS * 16,), jnp.float32),
        ],
        scratch_types=[
            pltpu.VMEM((IPW,), jnp.int32),
            pltpu.VMEM((IPW,), jnp.int32),
            pltpu.VMEM((IPW, 16), jnp.int32),
            pltpu.VMEM((IPW * 16,), jnp.int32),
            pltpu.VMEM((IPW * 16,), jnp.int32),
            pltpu.VMEM((IPW * 16,), jnp.int32),
            pltpu.VMEM((IPW * 16,), jnp.int32),
            pltpu.VMEM((32, 16), jnp.int32),
            pltpu.VMEM((32, 16), jnp.int32),
            pltpu.VMEM((32, 16), jnp.int32),
            pltpu.VMEM((32, 16), jnp.int32),
            pltpu.VMEM((512,), jnp.int32),
            pltpu.VMEM((512,), jnp.int32),
            pltpu.VMEM((512,), jnp.int32),
            pltpu.VMEM((512,), jnp.int32),
            pltpu.VMEM((512,), jnp.int32),
            pltpu.VMEM((512,), jnp.int32),
            pltpu.VMEM((512,), jnp.int32),
            pltpu.VMEM((512,), jnp.int32),
            pltpu.VMEM((IPW * S * HB,), jnp.float32),
            pltpu.VMEM((IPW * 16,), jnp.float32),
        ] + [pltpu.SemaphoreType.DMA] * 9,
    )


def _agg_tc_kernel(cnt_ref, m1_ref, rf_ref, w0_ref, b0_ref, w1_ref, b1_ref, out_ref):
    m = jnp.dot(rf_ref[...], w0_ref[...], preferred_element_type=jnp.float32) * (1.0 / 32.0)
    m48 = jnp.concatenate([m, jnp.zeros((HB - N_REL, H), jnp.float32)], axis=0)
    cnt = cnt_ref[...]
    new1 = jnp.dot(cnt, m48, preferred_element_type=jnp.float32) + b0_ref[...]
    new1 = jnp.maximum(new1, 0.0)
    blk = new1.shape[0] // S
    new1 = new1.reshape(blk, S, H) * m1_ref[...][:, :, None]
    pooled = jnp.sum(new1, axis=1) * (1.0 / S)
    out_ref[...] = jnp.dot(pooled, w1_ref[...], preferred_element_type=jnp.float32) + b1_ref[...]


def _selu(x):
    alpha = 1.6732632423543772848170429916717
    scale = 1.0507009873554804934193349852946
    return scale * jnp.where(x > 0, x, alpha * (jnp.exp(x) - 1.0))


def _head_tc_kernel(agg_ref, lw_ref, lb_ref, w1_ref, b1_ref, w2_ref, b2_ref,
                    w3_ref, b3_ref, w4_ref, b4_ref, t_ref, eps_ref,
                    scores_ref, loss_ref):
    agg = agg_ref[...]
    x0 = agg[:B, :]
    x1 = agg[B:, :]
    output = jnp.dot(jnp.concatenate([x0, x1], axis=1), lw_ref[...],
                     preferred_element_type=jnp.float32) + lb_ref[...]
    t = t_ref[...][:, None]
    mu_t = t * x1 + (1.0 - t) * x0
    xt = mu_t + SIGMA * eps_ref[...]
    ut = x1 - x0
    hcat = jnp.concatenate([xt, t], axis=1)
    v = _selu(jnp.dot(hcat, w1_ref[...], preferred_element_type=jnp.float32) + b1_ref[...])
    v = _selu(jnp.dot(v, w2_ref[...], preferred_element_type=jnp.float32) + b2_ref[...])
    v = _selu(jnp.dot(v, w3_ref[...], preferred_element_type=jnp.float32) + b3_ref[...])
    vt_pred = jnp.dot(v, w4_ref[...], preferred_element_type=jnp.float32) + b4_ref[...]
    loss_ref[0, 0] = jnp.mean((vt_pred - ut) ** 2)
    scores_ref[...] = output * vt_pred


@jax.jit
def kernel(entity_pairs, train_edges, labels, entity2edges, edge2entities,
           edge2relation, relation_features_base, agg0_W, agg0_b, agg1_W,
           agg1_b, lin_out_W, lin_out_b, mlp_W1, mlp_b1, mlp_W2, mlp_b2,
           mlp_W3, mlp_b3, mlp_W4, mlp_b4, t_rand, eps):
    del labels

    ents = jnp.concatenate([entity_pairs[:, 0], entity_pairs[:, 1]])
    te2 = jnp.concatenate([train_edges, train_edges])
    ent0 = edge2entities[:, 0]
    ent1 = edge2entities[:, 1]

    cnt_flat, m1_flat = _make_sc_call()(ents, te2, entity2edges, ent0, ent1,
                                        edge2relation)

    cnt = cnt_flat.reshape(NITEMS * S, HB)
    m1 = m1_flat.reshape(NITEMS, 16)

    blk = 256
    agg = pl.pallas_call(
        _agg_tc_kernel,
        grid=(NITEMS // blk,),
        in_specs=[
            pl.BlockSpec((blk * S, HB), lambda i: (i, 0)),
            pl.BlockSpec((blk, 16), lambda i: (i, 0)),
            pl.BlockSpec((N_REL, H), lambda i: (0, 0)),
            pl.BlockSpec((H, H), lambda i: (0, 0)),
            pl.BlockSpec((H,), lambda i: (0,)),
            pl.BlockSpec((H, N_REL), lambda i: (0, 0)),
            pl.BlockSpec((N_REL,), lambda i: (0,)),
        ],
        out_specs=pl.BlockSpec((blk, N_REL), lambda i: (i, 0)),
        out_shape=jax.ShapeDtypeStruct((NITEMS, N_REL), jnp.float32),
    )(cnt, m1, relation_features_base, agg0_W, agg0_b, agg1_W, agg1_b)

    scores, loss = pl.pallas_call(
        _head_tc_kernel,
        out_shape=(
            jax.ShapeDtypeStruct((B, N_REL), jnp.float32),
            jax.ShapeDtypeStruct((1, 1), jnp.float32),
        ),
        out_specs=(
            pl.BlockSpec(memory_space=pltpu.VMEM),
            pl.BlockSpec(memory_space=pltpu.SMEM),
        ),
    )(agg, lin_out_W, lin_out_b, mlp_W1, mlp_b1, mlp_W2, mlp_b2,
      mlp_W3, mlp_b3, mlp_W4, mlp_b4, t_rand, eps)

    return scores, loss[0, 0]


# speedup vs baseline: 22.5446x; 1.0281x over previous
import functools
import jax
import jax.numpy as jnp
from jax import lax
from jax.experimental import pallas as pl
from jax.experimental.pallas import tpu as pltpu
from jax.experimental.pallas import tpu_sc as plsc

N_REL = 32
N_ENT = 100000
N_EDGE = 1600000
S = 16
H = 128
B = 1024
W_CFM = 128
SIGMA = 0.1

NITEMS = 2 * B
NW = 32
IPW = NITEMS // NW
HB = 48


def _iota16():
    return lax.broadcasted_iota(jnp.int32, (16,), 0)


def _sc_kernel_body(ents_hbm, te_hbm, e2e_hbm, ent0_hbm, ent1_hbm, rel_hbm,
                    cnt_hbm, m1_hbm,
                    ents_v, te_v, e1, e1flat, ents2a, ents2b, r1,
                    e2a, e2b, e2c, e2d, tfa, tfb, tfc, tfd,
                    r2a, r2b, r2c, r2d, cntbuf, m1buf,
                    sem, sema, semb, semc, semd, semra, semrb, semrc, semrd):
    wid = lax.axis_index("s") * 2 + lax.axis_index("c")
    ibase = wid * IPW

    pltpu.sync_copy(ents_hbm.at[pl.ds(ibase, IPW)], ents_v)
    pltpu.sync_copy(te_hbm.at[pl.ds(ibase, IPW)], te_v)

    pltpu.async_copy(e2e_hbm.at[ents_v], e1, sem).wait()

    for j in range(IPW):
        e1flat[pl.ds(j * 16, 16)] = e1[j, :]

    handles = []
    for c in range(IPW * 16 // 128):
        idx = e1flat.at[pl.ds(c * 128, 128)]
        handles.append(pltpu.async_copy(ent0_hbm.at[idx],
                                        ents2a.at[pl.ds(c * 128, 128)], sem))
        handles.append(pltpu.async_copy(ent1_hbm.at[idx],
                                        ents2b.at[pl.ds(c * 128, 128)], sem))
        handles.append(pltpu.async_copy(rel_hbm.at[idx],
                                        r1.at[pl.ds(c * 128, 128)], sem))
    for h in handles:
        h.wait()

    def zero_body(z, _):
        cntbuf[pl.ds(z * 16, 16)] = jnp.zeros((16,), jnp.float32)
        return _
    lax.fori_loop(0, IPW * S * HB // 16, zero_body, 0)

    iota = _iota16()

    def fetch_e2(i, buf, bsem):
        h0 = pltpu.async_copy(e2e_hbm.at[ents2a.at[pl.ds(i * 16, 16)]],
                              buf.at[pl.ds(0, 16)], bsem)
        h1 = pltpu.async_copy(e2e_hbm.at[ents2b.at[pl.ds(i * 16, 16)]],
                              buf.at[pl.ds(16, 16)], bsem)
        return h0, h1

    def transpose_fire(buf, tf, r2, rsem):
        for k in range(32):
            e_bit = 0 if k < 16 else 1
            s_id = k % 16
            vec = buf[k, :]
            tgt = iota * 16 + (e_bit * 256 + s_id)
            plsc.store_scatter(tf, [tgt], vec)
        return [pltpu.async_copy(rel_hbm.at[tf.at[pl.ds(c * 128, 128)]],
                                 r2.at[pl.ds(c * 128, 128)], rsem)
                for c in range(4)]

    def hist(i, tf, r2, rh):
        te_b = plsc.load_gather(te_v, [iota * 0 + i])
        base = i * (S * HB)

        rvec1 = r1[pl.ds(i * 16, 16)]
        plsc.addupdate_scatter(cntbuf, [base + iota * HB + rvec1],
                               jnp.full((16,), 32.0, jnp.float32))

        e1vec = e1flat[pl.ds(i * 16, 16)]
        m1buf[pl.ds(i * 16, 16)] = jnp.where(e1vec != te_b, 1.0, 0.0).astype(jnp.float32)

        for h in rh:
            h.wait()
        for k in range(32):
            edges_vec = tf[pl.ds(k * 16, 16)]
            rvec = r2[pl.ds(k * 16, 16)]
            m = jnp.where(edges_vec != te_b, 1.0, 0.0).astype(jnp.float32)
            plsc.addupdate_scatter(cntbuf, [base + iota * HB + rvec], m)

    e2s = (e2a, e2b, e2c, e2d)
    tfs = (tfa, tfb, tfc, tfd)
    r2s = (r2a, r2b, r2c, r2d)
    esems = (sema, semb, semc, semd)
    rsems = (semra, semrb, semrc, semrd)

    def drain_e2(buf, bsem):
        for off in (0, 16):
            pltpu.make_async_copy(e2e_hbm.at[pl.ds(0, 16)],
                                  buf.at[pl.ds(off, 16)], bsem).wait()

    for b in range(4):
        fetch_e2(b, e2s[b], esems[b])

    def loop_body(j, _):
        i = j * 4
        rhs = []
        for b in range(4):
            drain_e2(e2s[b], esems[b])
            rhs.append(transpose_fire(e2s[b], tfs[b], r2s[b], rsems[b]))
        @pl.when(j + 1 < IPW // 4)
        def _prefetch():
            for b in range(4):
                fetch_e2(i + 4 + b, e2s[b], esems[b])
        for b in range(4):
            hist(i + b, tfs[b], r2s[b], rhs[b])
        return _

    lax.fori_loop(0, IPW // 4, loop_body, 0)

    pltpu.sync_copy(cntbuf, cnt_hbm.at[pl.ds(wid * (IPW * S * HB), IPW * S * HB)])
    pltpu.sync_copy(m1buf, m1_hbm.at[pl.ds(wid * (IPW * 16), IPW * 16)])


def _make_sc_call():
    mesh = plsc.VectorSubcoreMesh(core_axis_name="c", subcore_axis_name="s")
    return pl.kernel(
        _sc_kernel_body,
        mesh=mesh,
        compiler_params=pltpu.CompilerParams(needs_layout_passes=False,
                                             use_tc_tiling_on_sc=False),
        out_type=[
            jax.ShapeDtypeStruct((NITEMS * S * HB,), jnp.float32),
            jax.ShapeDtypeStruct((NITEMS * 16,), jnp.float32),
        ],
        scratch_types=[
            pltpu.VMEM((IPW,), jnp.int32),
            pltpu.VMEM((IPW,), jnp.int32),
            pltpu.VMEM((IPW, 16), jnp.int32),
            pltpu.VMEM((IPW * 16,), jnp.int32),
            pltpu.VMEM((IPW * 16,), jnp.int32),
            pltpu.VMEM((IPW * 16,), jnp.int32),
            pltpu.VMEM((IPW * 16,), jnp.int32),
            pltpu.VMEM((32, 16), jnp.int32),
            pltpu.VMEM((32, 16), jnp.int32),
            pltpu.VMEM((32, 16), jnp.int32),
            pltpu.VMEM((32, 16), jnp.int32),
            pltpu.VMEM((512,), jnp.int32),
            pltpu.VMEM((512,), jnp.int32),
            pltpu.VMEM((512,), jnp.int32),
            pltpu.VMEM((512,), jnp.int32),
            pltpu.VMEM((512,), jnp.int32),
            pltpu.VMEM((512,), jnp.int32),
            pltpu.VMEM((512,), jnp.int32),
            pltpu.VMEM((512,), jnp.int32),
            pltpu.VMEM((IPW * S * HB,), jnp.float32),
            pltpu.VMEM((IPW * 16,), jnp.float32),
        ] + [pltpu.SemaphoreType.DMA] * 9,
    )


def _agg_tc_kernel(cnt_ref, m1_ref, rf_ref, w0_ref, b0_ref, w1_ref, b1_ref, out_ref):
    m = jnp.dot(rf_ref[...], w0_ref[...], preferred_element_type=jnp.float32) * (1.0 / 32.0)
    m48 = jnp.concatenate([m, jnp.zeros((HB - N_REL, H), jnp.float32)], axis=0)
    cnt = cnt_ref[...]
    new1 = jnp.dot(cnt, m48, preferred_element_type=jnp.float32) + b0_ref[...]
    new1 = jnp.maximum(new1, 0.0)
    blk = new1.shape[0] // S
    new1 = new1.reshape(blk, S, H) * m1_ref[...][:, :, None]
    pooled = jnp.sum(new1, axis=1) * (1.0 / S)
    out_ref[...] = jnp.dot(pooled, w1_ref[...], preferred_element_type=jnp.float32) + b1_ref[...]


def _selu(x):
    alpha = 1.6732632423543772848170429916717
    scale = 1.0507009873554804934193349852946
    return scale * jnp.where(x > 0, x, alpha * (jnp.exp(x) - 1.0))


def _head_tc_kernel(agg_ref, lw_ref, lb_ref, w1_ref, b1_ref, w2_ref, b2_ref,
                    w3_ref, b3_ref, w4_ref, b4_ref, t_ref, eps_ref,
                    scores_ref, loss_ref):
    agg = agg_ref[...]
    x0 = agg[:B, :]
    x1 = agg[B:, :]
    output = jnp.dot(jnp.concatenate([x0, x1], axis=1), lw_ref[...],
                     preferred_element_type=jnp.float32) + lb_ref[...]
    t = t_ref[...][:, None]
    mu_t = t * x1 + (1.0 - t) * x0
    xt = mu_t + SIGMA * eps_ref[...]
    ut = x1 - x0
    hcat = jnp.concatenate([xt, t], axis=1)
    v = _selu(jnp.dot(hcat, w1_ref[...], preferred_element_type=jnp.float32) + b1_ref[...])
    v = _selu(jnp.dot(v, w2_ref[...], preferred_element_type=jnp.float32) + b2_ref[...])
    v = _selu(jnp.dot(v, w3_ref[...], preferred_element_type=jnp.float32) + b3_ref[...])
    vt_pred = jnp.dot(v, w4_ref[...], preferred_element_type=jnp.float32) + b4_ref[...]
    loss_ref[0, 0] = jnp.mean((vt_pred - ut) ** 2)
    scores_ref[...] = output * vt_pred


@jax.jit
def kernel(entity_pairs, train_edges, labels, entity2edges, edge2entities,
           edge2relation, relation_features_base, agg0_W, agg0_b, agg1_W,
           agg1_b, lin_out_W, lin_out_b, mlp_W1, mlp_b1, mlp_W2, mlp_b2,
           mlp_W3, mlp_b3, mlp_W4, mlp_b4, t_rand, eps):
    del labels

    ents = jnp.concatenate([entity_pairs[:, 0], entity_pairs[:, 1]])
    te2 = jnp.concatenate([train_edges, train_edges])
    ent0 = edge2entities[:, 0]
    ent1 = edge2entities[:, 1]

    cnt_flat, m1_flat = _make_sc_call()(ents, te2, entity2edges, ent0, ent1,
                                        edge2relation)

    cnt = cnt_flat.reshape(NITEMS * S, HB)
    m1 = m1_flat.reshape(NITEMS, 16)

    blk = 256
    agg = pl.pallas_call(
        _agg_tc_kernel,
        grid=(NITEMS // blk,),
        in_specs=[
            pl.BlockSpec((blk * S, HB), lambda i: (i, 0)),
            pl.BlockSpec((blk, 16), lambda i: (i, 0)),
            pl.BlockSpec((N_REL, H), lambda i: (0, 0)),
            pl.BlockSpec((H, H), lambda i: (0, 0)),
            pl.BlockSpec((H,), lambda i: (0,)),
            pl.BlockSpec((H, N_REL), lambda i: (0, 0)),
            pl.BlockSpec((N_REL,), lambda i: (0,)),
        ],
        out_specs=pl.BlockSpec((blk, N_REL), lambda i: (i, 0)),
        out_shape=jax.ShapeDtypeStruct((NITEMS, N_REL), jnp.float32),
    )(cnt, m1, relation_features_base, agg0_W, agg0_b, agg1_W, agg1_b)

    scores, loss = pl.pallas_call(
        _head_tc_kernel,
        out_shape=(
            jax.ShapeDtypeStruct((B, N_REL), jnp.float32),
            jax.ShapeDtypeStruct((1, 1), jnp.float32),
        ),
        out_specs=(
            pl.BlockSpec(memory_space=pltpu.VMEM),
            pl.BlockSpec(memory_space=pltpu.SMEM),
        ),
    )(agg, lin_out_W, lin_out_b, mlp_W1, mlp_b1, mlp_W2, mlp_b2,
      mlp_W3, mlp_b3, mlp_W4, mlp_b4, t_rand, eps)

    return scores, loss[0, 0]
